# baseline (speedup 1.0000x reference)
import jax
import jax.numpy as jnp
from jax.experimental import pallas as pl

N_ATOM_ = 10000
N_FRAG_ = 2000


def _mm_kernel(x_ref, w_ref, b_ref, o_ref):
    o_ref[...] = jnp.dot(x_ref[...], w_ref[...],
                         preferred_element_type=jnp.float32) + b_ref[...]


def _pallas_matmul_bias(x, W, b):
    M, K = x.shape
    N = W.shape[1]
    BM = 1000
    return pl.pallas_call(
        _mm_kernel,
        grid=(M // BM,),
        in_specs=[
            pl.BlockSpec((BM, K), lambda i: (i, 0)),
            pl.BlockSpec((K, N), lambda i: (0, 0)),
            pl.BlockSpec((1, N), lambda i: (0, 0)),
        ],
        out_specs=pl.BlockSpec((BM, N), lambda i: (i, 0)),
        out_shape=jax.ShapeDtypeStruct((M, N), jnp.float32),
    )(x, W, b.reshape(1, N))


def _gcn_conv(x, edge_index, W, b, num_nodes):
    xw = x @ W
    src = edge_index[0]
    dst = edge_index[1]
    deg = jnp.zeros((num_nodes,), jnp.float32).at[dst].add(1.0) + 1.0
    dinv = jax.lax.rsqrt(deg)
    norm = dinv[src] * dinv[dst]
    msg = xw[src] * norm[:, None]
    out = jnp.zeros((num_nodes, W.shape[1]), jnp.float32).at[dst].add(msg)
    out = out + xw * (dinv * dinv)[:, None]
    return out + b


def kernel(x, edge_index, frag_h, frag_edge_index, atom2u, s0,
           W_l1, b_l1, W_l2, b_l2, W_g1, b_g1, W_g2, b_g2, W_af, b_af, W_fa, b_fa):
    x_frag = frag_h @ W_l2 + b_l2
    x_atom = x @ W_l1 + b_l1 + s0
    x_atom = jax.nn.relu(_gcn_conv(x_atom, edge_index, W_g1, b_g1, N_ATOM_))
    x_atom_proj = _pallas_matmul_bias(x_atom, W_af, b_af)
    sums = jnp.zeros((N_FRAG_, x_atom_proj.shape[1]), jnp.float32).at[atom2u].add(x_atom_proj)
    cnt = jnp.zeros((N_FRAG_,), jnp.float32).at[atom2u].add(1.0)
    x_frag = x_frag + sums / jnp.maximum(cnt, 1.0)[:, None]
    x_frag = jax.nn.relu(_gcn_conv(x_frag, frag_edge_index, W_g2, b_g2, N_FRAG_))
    x_frag_proj = _pallas_matmul_bias(x_frag, W_fa, b_fa)
    x_atom = x_atom + x_frag_proj[atom2u]
    return (x_atom, x_frag)


# speedup vs baseline: 19.1130x; 19.1130x over previous
import functools

import jax
import jax.numpy as jnp
from jax import lax
from jax.experimental import pallas as pl
from jax.experimental.pallas import tpu as pltpu
from jax.experimental.pallas import tpu_sc as plsc

N_ATOM_ = 10000
N_FRAG_ = 2000
E_ATOM_ = 640000
E_FRAG_ = 16000
H_ = 128

NC = 2
NS = 16
NW = NC * NS
EPW = 128

N_AACC = 10112
N_FACC = 2048

NWIN_E = -(-(E_ATOM_ // NW) // EPW)
NWIN_A = -(-(N_ATOM_ // NW) // EPW)
NWIN_F = -(-(E_FRAG_ // NW) // EPW)

_vmesh = plsc.VectorSubcoreMesh(core_axis_name="c", subcore_axis_name="s")


def _make_hist(nwin, nacc):
    rpt = nacc // NS
    assert rpt % 8 == 0

    @functools.partial(
        pl.kernel,
        mesh=_vmesh,
        out_type=jax.ShapeDtypeStruct((NC * nacc,), jnp.float32),
        scratch_types=[
            pltpu.VMEM((EPW,), jnp.int32),
            pltpu.VMEM((EPW,), jnp.float32),
            pltpu.VMEM((rpt,), jnp.float32),
            pltpu.VMEM_SHARED((nacc,), jnp.float32),
        ],
    )
    def hist_kernel(dst_hbm, ones_hbm, zeros_hbm, out_hbm,
                    dstv, onesv, histv, hist):
        cid = lax.axis_index("c")
        sid = lax.axis_index("s")
        wid = cid * NS + sid

        pltpu.sync_copy(ones_hbm, onesv)

        @pl.when(sid == 0)
        def _():
            pltpu.sync_copy(zeros_hbm, hist)

        plsc.subcore_barrier()

        @pl.loop(0, nwin)
        def _(w):
            pltpu.sync_copy(dst_hbm.at[wid, w], dstv)
            pltpu.sync_copy(onesv, hist.at[dstv], add=True)

        plsc.subcore_barrier()
        pltpu.sync_copy(hist.at[pl.ds(sid * rpt, rpt)], histv)
        pltpu.sync_copy(histv, out_hbm.at[pl.ds(cid * nacc + sid * rpt, rpt)])

    return hist_kernel


def _make_scatter(nwin, nacc):
    rpt = nacc // NS
    assert rpt % 8 == 0

    @functools.partial(
        pl.kernel,
        mesh=_vmesh,
        out_type=jax.ShapeDtypeStruct((NC, nacc, H_), jnp.float32),
        scratch_types=[
            pltpu.VMEM((EPW,), jnp.int32),
            pltpu.VMEM((EPW,), jnp.int32),
            pltpu.VMEM((EPW, H_), jnp.float32),
            pltpu.VMEM_SHARED((nacc, H_), jnp.float32),
        ],
    )
    def scatter_kernel(table_hbm, src_hbm, dst_hbm, zeros_hbm, out_hbm,
                       srcv, dstv, rows, acc):
        cid = lax.axis_index("c")
        sid = lax.axis_index("s")
        wid = cid * NS + sid

        pltpu.sync_copy(
            zeros_hbm.at[pl.ds(sid * rpt, rpt)],
            acc.at[pl.ds(sid * rpt, rpt)],
        )
        plsc.subcore_barrier()

        @pl.loop(0, nwin)
        def _(w):
            pltpu.sync_copy(src_hbm.at[wid, w], srcv)
            pltpu.sync_copy(dst_hbm.at[wid, w], dstv)
            pltpu.sync_copy(table_hbm.at[srcv], rows)
            pltpu.sync_copy(rows, acc.at[dstv], add=True)

        plsc.subcore_barrier()
        pltpu.sync_copy(
            acc.at[pl.ds(sid * rpt, rpt)],
            out_hbm.at[cid, pl.ds(sid * rpt, rpt)],
        )

    return scatter_kernel


def _make_gather(nwin):
    nrow = NW * nwin * EPW

    @functools.partial(
        pl.kernel,
        mesh=_vmesh,
        out_type=jax.ShapeDtypeStruct((nrow, H_), jnp.float32),
        scratch_types=[
            pltpu.VMEM((EPW,), jnp.int32),
            pltpu.VMEM((EPW, H_), jnp.float32),
        ],
    )
    def gather_kernel(table_hbm, idx_hbm, out_hbm, idxv, rows):
        cid = lax.axis_index("c")
        sid = lax.axis_index("s")
        wid = cid * NS + sid

        @pl.loop(0, nwin)
        def _(w):
            pltpu.sync_copy(idx_hbm.at[wid, w], idxv)
            pltpu.sync_copy(table_hbm.at[idxv], rows)
            pltpu.sync_copy(rows, out_hbm.at[pl.ds(wid * (nwin * EPW) + w * EPW, EPW)])

    return gather_kernel


_hist_edges = _make_hist(NWIN_E, N_AACC)
_hist_atoms = _make_hist(NWIN_A, N_FACC)
_hist_fedges = _make_hist(NWIN_F, N_FACC)
_scatter_edges = _make_scatter(NWIN_E, N_AACC)
_scatter_atoms = _make_scatter(NWIN_A, N_FACC)
_scatter_fedges = _make_scatter(NWIN_F, N_FACC)
_gather_atoms = _make_gather(NWIN_A)


def _mm_kernel(x_ref, w_ref, b_ref, o_ref):
    o_ref[...] = jnp.dot(x_ref[...], w_ref[...],
                         preferred_element_type=jnp.float32) + b_ref[...]


def _pallas_matmul_bias(x, W, b):
    M, K = x.shape
    N = W.shape[1]
    BM = 1000
    return pl.pallas_call(
        _mm_kernel,
        grid=(M // BM,),
        in_specs=[
            pl.BlockSpec((BM, K), lambda i: (i, 0)),
            pl.BlockSpec((K, N), lambda i: (0, 0)),
            pl.BlockSpec((1, N), lambda i: (0, 0)),
        ],
        out_specs=pl.BlockSpec((BM, N), lambda i: (i, 0)),
        out_shape=jax.ShapeDtypeStruct((M, N), jnp.float32),
    )(x, W, b.reshape(1, N))


def _pad_plane(ids, nwin, fill):
    total = NW * nwin * EPW
    ids = jnp.pad(ids, (0, total - ids.shape[0]), constant_values=fill)
    return ids.reshape(NW, nwin, EPW)


def kernel(x, edge_index, frag_h, frag_edge_index, atom2u, s0,
           W_l1, b_l1, W_l2, b_l2, W_g1, b_g1, W_g2, b_g2, W_af, b_af, W_fa, b_fa):
    esrc_p = _pad_plane(edge_index[0], NWIN_E, 0)
    edst_p = _pad_plane(edge_index[1], NWIN_E, N_ATOM_)
    a2u_p = _pad_plane(atom2u, NWIN_A, N_FRAG_)
    a2u_g = _pad_plane(atom2u, NWIN_A, 0)
    aidx_p = _pad_plane(jnp.arange(N_ATOM_, dtype=jnp.int32), NWIN_A, 0)
    fsrc_p = _pad_plane(frag_edge_index[0], NWIN_F, 0)
    fdst_p = _pad_plane(frag_edge_index[1], NWIN_F, N_FRAG_)

    ones128 = jnp.ones((EPW,), jnp.float32)
    zeros_a1 = jnp.zeros((N_AACC,), jnp.float32)
    zeros_f1 = jnp.zeros((N_FACC,), jnp.float32)
    zeros_a2 = jnp.zeros((N_AACC, H_), jnp.float32)
    zeros_f2 = jnp.zeros((N_FACC, H_), jnp.float32)

    deg_p = _hist_edges(edst_p, ones128, zeros_a1).reshape(NC, N_AACC)
    deg = deg_p[0, :N_ATOM_] + deg_p[1, :N_ATOM_] + 1.0
    dinv = lax.rsqrt(deg)

    cnt_p = _hist_atoms(a2u_p, ones128, zeros_f1).reshape(NC, N_FACC)
    cnt = cnt_p[0, :N_FRAG_] + cnt_p[1, :N_FRAG_]

    fdeg_p = _hist_fedges(fdst_p, ones128, zeros_f1).reshape(NC, N_FACC)
    fdeg = fdeg_p[0, :N_FRAG_] + fdeg_p[1, :N_FRAG_] + 1.0
    fdinv = lax.rsqrt(fdeg)

    x_frag = frag_h @ W_l2 + b_l2
    x_atom_lin = x @ W_l1 + b_l1 + s0

    xs = (x_atom_lin @ W_g1) * dinv[:, None]
    acc_p = _scatter_edges(xs, esrc_p, edst_p, zeros_a2)
    acc = acc_p[0, :N_ATOM_] + acc_p[1, :N_ATOM_]
    x_atom = jax.nn.relu((acc + xs) * dinv[:, None] + b_g1)

    x_atom_proj = _pallas_matmul_bias(x_atom, W_af, b_af)
    sums_p = _scatter_atoms(x_atom_proj, aidx_p, a2u_p, zeros_f2)
    sums = sums_p[0, :N_FRAG_] + sums_p[1, :N_FRAG_]
    x_frag = x_frag + sums / jnp.maximum(cnt, 1.0)[:, None]

    fxs = (x_frag @ W_g2) * fdinv[:, None]
    facc_p = _scatter_fedges(fxs, fsrc_p, fdst_p, zeros_f2)
    facc = facc_p[0, :N_FRAG_] + facc_p[1, :N_FRAG_]
    x_frag = jax.nn.relu((facc + fxs) * fdinv[:, None] + b_g2)

    x_frag_proj = _pallas_matmul_bias(x_frag, W_fa, b_fa)
    bcast = _gather_atoms(x_frag_proj, a2u_g)
    x_atom = x_atom + bcast[:N_ATOM_]
    return (x_atom, x_frag)
